# baseline (speedup 1.0000x reference)
import jax
import jax.numpy as jnp
from jax.experimental import pallas as pl
from jax.experimental.pallas import tpu as pltpu

BLK = 256


def _wh_kernel(h_ref, w_ref, ai_ref, aj_ref, whb_ref, fi_ref, fjt_ref):
    wh = jax.lax.dot_general(
        h_ref[...], w_ref[...], (((1,), (1,)), ((), ())),
        preferred_element_type=jnp.float32)
    fi_ref[...] = jnp.dot(wh, ai_ref[...], preferred_element_type=jnp.float32)
    fjt_ref[...] = jax.lax.dot_general(
        aj_ref[...], wh, (((1,), (1,)), ((), ())),
        preferred_element_type=jnp.float32)
    whb_ref[...] = wh.astype(jnp.bfloat16)


def _attn_kernel(blkf_ref, blkl_ref, ac_ref, wh_ref, fi_ref, fjt_ref,
                 indr_ref, indc_ref, out_ref, acc_ref, m_ref, s_ref, cnt_ref):
    r = pl.program_id(0)
    nb = pl.num_programs(0)
    acc_ref[...] = jnp.zeros_like(acc_ref)
    m_ref[...] = jnp.full_like(m_ref, -1e30)
    s_ref[...] = jnp.zeros_like(s_ref)
    cnt_ref[...] = jnp.zeros_like(cnt_ref)
    r_first = blkf_ref[r]
    r_last = blkl_ref[r]
    fi = fi_ref[...]
    ids_r = indr_ref[...]
    ac = ac_ref[0, 0]

    def body(c, carry):
        @pl.when((blkf_ref[c] <= r_last) & (blkl_ref[c] >= r_first))
        def _process():
            e = fi + fjt_ref[:, pl.ds(c * BLK, BLK)] + ac
            e = jnp.where(e >= 0, e, 0.1 * e)
            mask = ids_r == indc_ref[:, pl.ds(c * BLK, BLK)]
            e = jnp.where(mask, e, -1e9)
            m_old = m_ref[...]
            m_new = jnp.maximum(m_old, jnp.max(e, axis=1, keepdims=True))
            p = jnp.where(mask, jnp.exp(e - m_new), 0.0)
            scale = jnp.exp(m_old - m_new)
            whc = wh_ref[pl.ds(c * BLK, BLK), :]
            acc_ref[...] = acc_ref[...] * scale + jax.lax.dot_general(
                p.astype(jnp.bfloat16), whc, (((1,), (0,)), ((), ())),
                preferred_element_type=jnp.float32)
            s_ref[...] = s_ref[...] * scale + jnp.sum(p, axis=1, keepdims=True)
            m_ref[...] = m_new
            cnt_ref[...] = cnt_ref[...] + jnp.sum(
                mask.astype(jnp.int32), axis=1, keepdims=True)
        return carry

    jax.lax.fori_loop(0, nb, body, 0)
    out_ref[...] = jnp.where(cnt_ref[...] > 1,
                             acc_ref[...] / s_ref[...], 0.0)


def kernel(h, ind_id, W, att_w):
    n, hid = h.shape
    a = att_w[0]
    ai = a[:hid].reshape(hid, 1)
    aj = a[hid:2 * hid].reshape(1, hid)
    ac = a[2 * hid].reshape(1, 1)

    whb, fi, fjt = pl.pallas_call(
        _wh_kernel,
        out_shape=(
            jax.ShapeDtypeStruct((n, hid), jnp.bfloat16),
            jax.ShapeDtypeStruct((n, 1), jnp.float32),
            jax.ShapeDtypeStruct((1, n), jnp.float32),
        ),
    )(h.astype(jnp.bfloat16), W.astype(jnp.bfloat16), ai, aj)

    indr = ind_id.reshape(n, 1)
    indc = ind_id.reshape(1, n)
    blk_first = ind_id[0::BLK]
    blk_last = ind_id[BLK - 1::BLK]

    out = pl.pallas_call(
        _attn_kernel,
        grid=(n // BLK,),
        in_specs=[
            pl.BlockSpec(memory_space=pltpu.SMEM),
            pl.BlockSpec(memory_space=pltpu.SMEM),
            pl.BlockSpec(memory_space=pltpu.SMEM),
            pl.BlockSpec((n, hid), lambda r: (0, 0)),
            pl.BlockSpec((BLK, 1), lambda r: (r, 0)),
            pl.BlockSpec((1, n), lambda r: (0, 0)),
            pl.BlockSpec((BLK, 1), lambda r: (r, 0)),
            pl.BlockSpec((1, n), lambda r: (0, 0)),
        ],
        out_specs=pl.BlockSpec((BLK, hid), lambda r: (r, 0)),
        out_shape=jax.ShapeDtypeStruct((n, hid), jnp.float32),
        scratch_shapes=[
            pltpu.VMEM((BLK, hid), jnp.float32),
            pltpu.VMEM((BLK, 1), jnp.float32),
            pltpu.VMEM((BLK, 1), jnp.float32),
            pltpu.VMEM((BLK, 1), jnp.int32),
        ],
    )(blk_first, blk_last, ac, whb, fi, fjt, indr, indc)
    return out


# speedup vs baseline: 1.0497x; 1.0497x over previous
import jax
import jax.numpy as jnp
from jax.experimental import pallas as pl
from jax.experimental.pallas import tpu as pltpu

BLK = 256


def _egat_kernel(blkf_ref, blkl_ref, ac_ref, h_ref, w_ref, ai_ref, aj_ref,
                 indr_ref, indc_ref, out_ref,
                 whb_ref, fi_ref, fjt_ref, acc_ref, m_ref, s_ref, cnt_ref):
    wh = jax.lax.dot_general(
        h_ref[...], w_ref[...], (((1,), (1,)), ((), ())),
        preferred_element_type=jnp.float32)
    whb_ref[...] = wh.astype(jnp.bfloat16)
    fi_ref[...] = jnp.dot(
        wh, ai_ref[...], preferred_element_type=jnp.float32) + ac_ref[0, 0]
    fjt_ref[...] = jax.lax.dot_general(
        aj_ref[...], wh, (((1,), (1,)), ((), ())),
        preferred_element_type=jnp.float32)
    nb = indc_ref.shape[1] // BLK

    def row_body(r, carry_r):
        fi_r = fi_ref[pl.ds(r * BLK, BLK), :]
        ids_r = indr_ref[pl.ds(r * BLK, BLK), :]
        r_first = blkf_ref[r]
        r_last = blkl_ref[r]
        acc_ref[...] = jnp.zeros_like(acc_ref)
        m_ref[...] = jnp.full_like(m_ref, -1e30)
        s_ref[...] = jnp.zeros_like(s_ref)
        cnt_ref[...] = jnp.zeros_like(cnt_ref)

        def col_body(c, carry_c):
            @pl.when((blkf_ref[c] <= r_last) & (blkl_ref[c] >= r_first))
            def _process():
                e0 = fi_r + fjt_ref[:, pl.ds(c * BLK, BLK)]
                e = jnp.maximum(e0, 0.1 * e0)
                mask = ids_r == indc_ref[:, pl.ds(c * BLK, BLK)]
                em = jnp.where(mask, e, -1e9)
                m_old = m_ref[...]
                m_new = jnp.maximum(m_old,
                                    jnp.max(em, axis=1, keepdims=True))
                p = jnp.exp(em - m_new)
                scale = jnp.exp(m_old - m_new)
                whc = whb_ref[pl.ds(c * BLK, BLK), :]
                acc_ref[...] = acc_ref[...] * scale + jax.lax.dot_general(
                    p.astype(jnp.bfloat16), whc, (((1,), (0,)), ((), ())),
                    preferred_element_type=jnp.float32)
                s_ref[...] = (s_ref[...] * scale
                              + jnp.sum(p, axis=1, keepdims=True))
                m_ref[...] = m_new
                cnt_ref[...] = cnt_ref[...] + jnp.sum(
                    mask.astype(jnp.int32), axis=1, keepdims=True)
            return carry_c

        jax.lax.fori_loop(0, nb, col_body, 0)
        out_ref[pl.ds(r * BLK, BLK), :] = jnp.where(
            cnt_ref[...] > 1, acc_ref[...] / s_ref[...], 0.0)
        return carry_r

    jax.lax.fori_loop(0, nb, row_body, 0)


def kernel(h, ind_id, W, att_w):
    n, hid = h.shape
    a = att_w[0]
    ai = a[:hid].reshape(hid, 1)
    aj = a[hid:2 * hid].reshape(1, hid)
    ac = a[2 * hid].reshape(1, 1)
    indr = ind_id.reshape(n, 1)
    indc = ind_id.reshape(1, n)
    blk_first = ind_id[0::BLK]
    blk_last = ind_id[BLK - 1::BLK]

    vmem = pl.BlockSpec(memory_space=pltpu.VMEM)
    smem = pl.BlockSpec(memory_space=pltpu.SMEM)
    out = pl.pallas_call(
        _egat_kernel,
        in_specs=[smem, smem, smem, vmem, vmem, vmem, vmem, vmem, vmem],
        out_specs=vmem,
        out_shape=jax.ShapeDtypeStruct((n, hid), jnp.float32),
        scratch_shapes=[
            pltpu.VMEM((n, hid), jnp.bfloat16),
            pltpu.VMEM((n, 1), jnp.float32),
            pltpu.VMEM((1, n), jnp.float32),
            pltpu.VMEM((BLK, hid), jnp.float32),
            pltpu.VMEM((BLK, 1), jnp.float32),
            pltpu.VMEM((BLK, 1), jnp.float32),
            pltpu.VMEM((BLK, 1), jnp.int32),
        ],
    )(blk_first, blk_last, ac, h.astype(jnp.bfloat16),
      W.astype(jnp.bfloat16), ai, aj, indr, indc)
    return out


# speedup vs baseline: 1.6925x; 1.6124x over previous
import jax
import jax.numpy as jnp
from jax.experimental import pallas as pl
from jax.experimental.pallas import tpu as pltpu

BLK = 256


def _egat_kernel(ind_smem_ref, att_smem_ref, h_ref, w_ref, att_ref, ai_ref,
                 indc_ref, out_ref,
                 whb_ref, fi_ref, fjt_ref, idsr_ref,
                 acc_ref, m_ref, s_ref, cnt_ref):
    n, hid = h_ref.shape
    nb = n // BLK
    hb = h_ref[...].astype(jnp.bfloat16)
    wb = w_ref[...].astype(jnp.bfloat16)
    wh = jax.lax.dot_general(
        hb, wb, (((1,), (1,)), ((), ())), preferred_element_type=jnp.float32)
    whb_ref[...] = wh.astype(jnp.bfloat16)
    ac = att_smem_ref[0, 2 * hid]
    aj_row = att_ref[:, hid:2 * hid]
    fi_ref[...] = jnp.dot(
        wh, ai_ref[...], preferred_element_type=jnp.float32) + ac
    fjt_ref[...] = jax.lax.dot_general(
        aj_row, wh, (((1,), (1,)), ((), ())),
        preferred_element_type=jnp.float32)
    idsr_ref[...] = indc_ref[...].reshape(n, 1)

    def row_body(r, carry_r):
        fi_r = fi_ref[pl.ds(r * BLK, BLK), :]
        ids_r = idsr_ref[pl.ds(r * BLK, BLK), :]
        r_first = ind_smem_ref[r * BLK]
        r_last = ind_smem_ref[r * BLK + BLK - 1]
        acc_ref[...] = jnp.zeros_like(acc_ref)
        m_ref[...] = jnp.full_like(m_ref, -1e30)
        s_ref[...] = jnp.zeros_like(s_ref)
        cnt_ref[...] = jnp.zeros_like(cnt_ref)

        for c in range(nb):
            @pl.when((ind_smem_ref[c * BLK] <= r_last)
                     & (ind_smem_ref[c * BLK + BLK - 1] >= r_first))
            def _process():
                e0 = fi_r + fjt_ref[:, c * BLK:(c + 1) * BLK]
                e = jnp.maximum(e0, 0.1 * e0)
                mask = ids_r == indc_ref[:, c * BLK:(c + 1) * BLK]
                em = jnp.where(mask, e, -1e9)
                m_old = m_ref[...]
                m_new = jnp.maximum(m_old,
                                    jnp.max(em, axis=1, keepdims=True))
                p = jnp.exp(em - m_new)
                scale = jnp.exp(m_old - m_new)
                whc = whb_ref[c * BLK:(c + 1) * BLK, :]
                acc_ref[...] = acc_ref[...] * scale + jax.lax.dot_general(
                    p.astype(jnp.bfloat16), whc, (((1,), (0,)), ((), ())),
                    preferred_element_type=jnp.float32)
                s_ref[...] = (s_ref[...] * scale
                              + jnp.sum(p, axis=1, keepdims=True))
                m_ref[...] = m_new
                cnt_ref[...] = cnt_ref[...] + jnp.sum(
                    mask.astype(jnp.int32), axis=1, keepdims=True)

        out_ref[pl.ds(r * BLK, BLK), :] = jnp.where(
            cnt_ref[...] > 1, acc_ref[...] / s_ref[...], 0.0)
        return carry_r

    jax.lax.fori_loop(0, nb, row_body, 0)


def kernel(h, ind_id, W, att_w):
    n, hid = h.shape
    indc = ind_id.reshape(1, n)
    ai = att_w[0, :hid].reshape(hid, 1)

    vmem = pl.BlockSpec(memory_space=pltpu.VMEM)
    smem = pl.BlockSpec(memory_space=pltpu.SMEM)
    out = pl.pallas_call(
        _egat_kernel,
        in_specs=[smem, smem, vmem, vmem, vmem, vmem, vmem],
        out_specs=vmem,
        out_shape=jax.ShapeDtypeStruct((n, hid), jnp.float32),
        scratch_shapes=[
            pltpu.VMEM((n, hid), jnp.bfloat16),
            pltpu.VMEM((n, 1), jnp.float32),
            pltpu.VMEM((1, n), jnp.float32),
            pltpu.VMEM((n, 1), jnp.int32),
            pltpu.VMEM((BLK, hid), jnp.float32),
            pltpu.VMEM((BLK, 1), jnp.float32),
            pltpu.VMEM((BLK, 1), jnp.float32),
            pltpu.VMEM((BLK, 1), jnp.int32),
        ],
    )(ind_id, att_w, h, W, att_w, ai, indc)
    return out


# speedup vs baseline: 2.0868x; 1.2329x over previous
import jax
import jax.numpy as jnp
from jax.experimental import pallas as pl
from jax.experimental.pallas import tpu as pltpu

BLK = 256


def _egat_kernel(ind_smem_ref, att_smem_ref, h_ref, w_ref, att_ref, ai_ref,
                 indc_ref, out_ref,
                 whb_ref, fi_ref, fjt_ref, idsr_ref, sing_ref,
                 acc_ref, s_ref):
    n, hid = h_ref.shape
    nb = n // BLK
    hb = h_ref[...].astype(jnp.bfloat16)
    wb = w_ref[...].astype(jnp.bfloat16)
    wh = jax.lax.dot_general(
        hb, wb, (((1,), (1,)), ((), ())), preferred_element_type=jnp.float32)
    whb_ref[...] = wh.astype(jnp.bfloat16)
    ac = att_smem_ref[0, 2 * hid]
    aj_row = att_ref[:, hid:2 * hid]
    fi = jnp.dot(wh, ai_ref[...], preferred_element_type=jnp.float32) + ac
    fi_ref[...] = fi
    fjt = jax.lax.dot_general(
        aj_row, wh, (((1,), (1,)), ((), ())),
        preferred_element_type=jnp.float32)
    fjt_ref[...] = fjt

    ids_c = indc_ref[...]
    idsr_ref[...] = ids_c.reshape(n, 1)
    sing = ((ids_c != pltpu.roll(ids_c, 1, axis=1))
            & (ids_c != pltpu.roll(ids_c, n - 1, axis=1)))
    sing_ref[...] = sing.astype(jnp.int32).reshape(n, 1)

    m0 = jnp.max(fi) + jnp.max(fjt)
    m_top = jnp.maximum(m0, 0.1 * m0)

    def row_body(r, carry_r):
        fi_r = fi_ref[pl.ds(r * BLK, BLK), :]
        ids_r = idsr_ref[pl.ds(r * BLK, BLK), :]
        r_first = ind_smem_ref[r * BLK]
        r_last = ind_smem_ref[r * BLK + BLK - 1]
        acc_ref[...] = jnp.zeros_like(acc_ref)
        s_ref[...] = jnp.zeros_like(s_ref)

        for c in range(nb):
            @pl.when((ind_smem_ref[c * BLK] <= r_last)
                     & (ind_smem_ref[c * BLK + BLK - 1] >= r_first))
            def _process():
                e0 = fi_r + fjt_ref[:, c * BLK:(c + 1) * BLK]
                e = jnp.maximum(e0, 0.1 * e0)
                mask = ids_r == indc_ref[:, c * BLK:(c + 1) * BLK]
                p = jnp.where(mask, jnp.exp(e - m_top), 0.0)
                whc = whb_ref[c * BLK:(c + 1) * BLK, :]
                acc_ref[...] = acc_ref[...] + jax.lax.dot_general(
                    p.astype(jnp.bfloat16), whc, (((1,), (0,)), ((), ())),
                    preferred_element_type=jnp.float32)
                s_ref[...] = s_ref[...] + jnp.sum(p, axis=1, keepdims=True)

        sing_r = sing_ref[pl.ds(r * BLK, BLK), :]
        out_ref[pl.ds(r * BLK, BLK), :] = jnp.where(
            sing_r == 0, acc_ref[...] / s_ref[...], 0.0)
        return carry_r

    jax.lax.fori_loop(0, nb, row_body, 0)


def kernel(h, ind_id, W, att_w):
    n, hid = h.shape
    indc = ind_id.reshape(1, n)
    ai = att_w[0, :hid].reshape(hid, 1)

    vmem = pl.BlockSpec(memory_space=pltpu.VMEM)
    smem = pl.BlockSpec(memory_space=pltpu.SMEM)
    out = pl.pallas_call(
        _egat_kernel,
        in_specs=[smem, smem, vmem, vmem, vmem, vmem, vmem],
        out_specs=vmem,
        out_shape=jax.ShapeDtypeStruct((n, hid), jnp.float32),
        scratch_shapes=[
            pltpu.VMEM((n, hid), jnp.bfloat16),
            pltpu.VMEM((n, 1), jnp.float32),
            pltpu.VMEM((1, n), jnp.float32),
            pltpu.VMEM((n, 1), jnp.int32),
            pltpu.VMEM((n, 1), jnp.int32),
            pltpu.VMEM((BLK, hid), jnp.float32),
            pltpu.VMEM((BLK, 1), jnp.float32),
        ],
    )(ind_id, att_w, h, W, att_w, ai, indc)
    return out
